# Initial kernel scaffold; baseline (speedup 1.0000x reference)
import jax
import jax.numpy as jnp
from jax.experimental import pallas as pl


def kernel(entity_emb, relation_emb, fc_w, w_q, w_h_entity, w_h_dialogue, out_w_init, out_w_q, dialogue_context, node_ids, edge_types, edge_index, seed_set):
    raise NotImplementedError("write your pallas kernel here")


# speedup vs baseline: 3.1436x; 3.1436x over previous
import functools

import jax
import jax.numpy as jnp
from jax import lax
from jax.experimental import pallas as pl
from jax.experimental.pallas import tpu as pltpu
from jax.experimental.pallas import tpu_sc as plsc

N = 10000
E = 160000
D = 256
H = 4
NREL = 50
NEG = 0.01
NT = N + NREL
TPAD = 10240
NW = 32
NB = 16
EPB = 10112
EP = EPB * NB
BPT = EPB // NW
NPP = N + 48
TW = NPP * H
ACCW = 10112
PCH = NPP + 192
ESP = H * D + 128
BIG = -1e30


def _mm(a, b, ca, cb):
    return lax.dot_general(a, b, (((ca,), (cb,)), ((), ())),
                           preferred_element_type=jnp.float32)


def _proj_body(x_ref, w_ref, o_ref):
    o_ref[...] = _mm(x_ref[...], w_ref[...], 1, 1)


def _tc_project(x, w):
    R = x.shape[0]
    return pl.pallas_call(
        _proj_body,
        grid=(R // 1024,),
        in_specs=[pl.BlockSpec((1024, D), lambda i: (i, 0)),
                  pl.BlockSpec((D, D), lambda i: (0, 0))],
        out_specs=pl.BlockSpec((1024, D), lambda i: (i, 0)),
        out_shape=jax.ShapeDtypeStruct((R, D), jnp.float32),
    )(x, w)


def _edst_body(x_ref, wq_ref, o_ref):
    x = x_ref[...]
    for h in range(H):
        o_ref[:, h * D:(h + 1) * D] = _mm(x, wq_ref[h], 1, 0)


def _tc_edst(ent, wq, rows_out):
    return pl.pallas_call(
        _edst_body,
        grid=(10,),
        in_specs=[pl.BlockSpec((1024, D), lambda i: (i, 0)),
                  pl.BlockSpec((H, D, D), lambda i: (0, 0, 0))],
        out_specs=pl.BlockSpec((1024, H * D), lambda i: (i, 0)),
        out_shape=jax.ShapeDtypeStruct((rows_out, H * D), jnp.float32),
    )(ent, wq)


def _esrcp_body(x_ref, wq_ref, fr_ref, o_ref):
    x = x_ref[...]
    for h in range(H):
        o_ref[:, h * D:(h + 1) * D] = _mm(x, wq_ref[h], 1, 0)
    o_ref[:, H * D:] = _mm(x, fr_ref[...], 1, 1)


def _tc_esrcp(ent, wq, frel_pad):
    return pl.pallas_call(
        _esrcp_body,
        grid=(10,),
        in_specs=[pl.BlockSpec((1024, D), lambda i: (i, 0)),
                  pl.BlockSpec((H, D, D), lambda i: (0, 0, 0)),
                  pl.BlockSpec((128, D), lambda i: (0, 0))],
        out_specs=pl.BlockSpec((1024, ESP), lambda i: (i, 0)),
        out_shape=jax.ShapeDtypeStruct((N + NB, ESP), jnp.float32),
    )(ent, wq, frel_pad)


def _merge_body(m_ref, s_ref, mg_ref, inv_ref):
    m = m_ref[...]
    s = s_ref[...]
    mg = jnp.max(m, axis=0, keepdims=True)
    sg = jnp.sum(s * jnp.exp(m - mg), axis=0, keepdims=True)
    mg_ref[...] = mg
    inv_ref[...] = 1.0 / (sg + 1e-16)


def _tc_merge(mtab, stab):
    return pl.pallas_call(
        _merge_body,
        in_specs=[pl.BlockSpec((NW, TW), lambda: (0, 0)),
                  pl.BlockSpec((NW, TW), lambda: (0, 0))],
        out_specs=[pl.BlockSpec((1, TW), lambda: (0, 0)),
                   pl.BlockSpec((1, TW), lambda: (0, 0))],
        out_shape=[jax.ShapeDtypeStruct((1, TW), jnp.float32),
                   jax.ShapeDtypeStruct((1, TW), jnp.float32)],
    )(mtab, stab)


def _iter_body(p_ref, wh_ref, dctx_ref, whd_ref, wq_ref, ef_ref, edst_ref):
    agg = p_ref[0] + p_ref[1]
    dch = _mm(dctx_ref[...], whd_ref[...], 1, 0)
    ef = _mm(agg, wh_ref[...], 1, 0) + dch
    ef_ref[...] = ef
    if edst_ref is not None:
        for h in range(H):
            edst_ref[:, h * D:(h + 1) * D] = _mm(ef, wq_ref[h], 1, 0)


def _tc_iter(partial, wh, dctx, whd, wq, with_q):
    if with_q:
        body = _iter_body
        out_specs = [pl.BlockSpec((1024, D), lambda i: (i, 0)),
                     pl.BlockSpec((1024, H * D), lambda i: (i, 0))]
        out_shape = [jax.ShapeDtypeStruct((N + NB, D), jnp.float32),
                     jax.ShapeDtypeStruct((N + NB, H * D), jnp.float32)]
    else:
        def body(a, b, c, d, e, f):
            return _iter_body(a, b, c, d, e, f, None)
        out_specs = [pl.BlockSpec((1024, D), lambda i: (i, 0))]
        out_shape = [jax.ShapeDtypeStruct((N + NB, D), jnp.float32)]
    return pl.pallas_call(
        body,
        grid=(10,),
        in_specs=[
            pl.BlockSpec((2, 1024, H * D), lambda i: (0, i, 0)),
            pl.BlockSpec((H * D, D), lambda i: (0, 0)),
            pl.BlockSpec((1, D), lambda i: (0, 0)),
            pl.BlockSpec((D, D), lambda i: (0, 0)),
            pl.BlockSpec((H, D, D), lambda i: (0, 0, 0)),
        ],
        out_specs=out_specs,
        out_shape=out_shape,
    )(partial, wh, dctx, whd, wq)


def _csa_body(ef_ref, dctx_ref, owi_ref, seeds_ref, o_ref):
    dcv = _mm(dctx_ref[...], owi_ref[...], 1, 0)
    csa = _mm(dcv, ef_ref[...], 1, 1)
    ids = lax.broadcasted_iota(jnp.int32, (1, N), 1)
    for i in range(32):
        csa = csa + jnp.where(ids == seeds_ref[i], 10000.0, 0.0)
    csa = csa - 10000.0
    m = jnp.max(csa)
    ex = jnp.exp(csa - m)
    o_ref[...] = ex / jnp.sum(ex)


def _tc_csa(ef, dctx, owi, seeds):
    return pl.pallas_call(
        _csa_body,
        in_specs=[
            pl.BlockSpec((N, D), lambda: (0, 0)),
            pl.BlockSpec((1, D), lambda: (0, 0)),
            pl.BlockSpec((D, D), lambda: (0, 0)),
            pl.BlockSpec(memory_space=pltpu.SMEM),
        ],
        out_specs=pl.BlockSpec((1, N), lambda: (0, 0)),
        out_shape=jax.ShapeDtypeStruct((1, N), jnp.float32),
    )(ef, dctx, owi, seeds)


def _comb_body(ap_ref, o_ref):
    o_ref[...] = jnp.sum(ap_ref[...], axis=0, keepdims=True)


def _tc_combine(apart):
    return pl.pallas_call(
        _comb_body,
        in_specs=[pl.BlockSpec((NW, ACCW), lambda: (0, 0))],
        out_specs=pl.BlockSpec((1, ACCW), lambda: (0, 0)),
        out_shape=jax.ShapeDtypeStruct((1, ACCW), jnp.float32),
    )(apart)


def _mesh():
    return plsc.VectorSubcoreMesh(core_axis_name="c", subcore_axis_name="s")


def _lane():
    return lax.broadcasted_iota(jnp.int32, (NB,), 0)


def _bcast(v, j):
    return v[jnp.full((NB,), j, jnp.int32)]


def _allsum(v):
    lane = _lane()
    for sh in (8, 4, 2, 1):
        v = v + v[lane ^ sh]
    return v


def _prefix_ms(keys, lo):
    lane = _lane()
    m = lo
    s = jnp.ones((NB,), jnp.float32)
    for sh in (1, 2, 4, 8):
        idx = jnp.maximum(lane - sh, 0)
        pm = m[idx]
        ps = s[idx]
        same = (keys[idx] == keys) & (lane >= sh)
        mn = jnp.maximum(m, jnp.where(same, pm, BIG))
        s = s * jnp.exp(m - mn) + jnp.where(same, ps * jnp.exp(pm - mn), 0.0)
        m = mn
    return m, s


def _runend(keys):
    lane = _lane()
    nxt = keys[jnp.minimum(lane + 1, NB - 1)]
    return (keys != nxt) | (lane == NB - 1)


def _wid():
    return lax.axis_index("s") * 2 + lax.axis_index("c")


def _dots(rows_v, q_v, j, qoff, extra):
    acc = [jnp.zeros((NB,), jnp.float32) for _ in range(H)]
    for k in range(D // NB):
        sl = pl.ds(k * NB, NB)
        u = rows_v[2 * j, sl] + rows_v[2 * j + 1, sl]
        for h in range(H):
            acc[h] = acc[h] + u * q_v[j, pl.ds(qoff + h * D + k * NB, NB)]
    return [_allsum(a) + (0.0 if extra is None else extra) for a in acc]


def _ainf_body(table_hbm, edst_hbm, idx_hbm, key_hbm, lo_hbm, mt_hbm, st_hbm,
               idx_v, key_v, rows_v, ed_v, lo_v, mt_v, st_v, sem):
    w = _wid()
    lane = _lane()

    def zinit(i, _):
        sl = pl.ds(i * NB, NB)
        mt_v[sl] = jnp.full((NB,), BIG, jnp.float32)
        st_v[sl] = jnp.zeros((NB,), jnp.float32)
        return 0

    lax.fori_loop(0, TW // NB, zinit, 0)

    def batch(b, _):
        r = w * BPT + b
        pltpu.sync_copy(idx_hbm.at[pl.ds(r, 1)], idx_v)
        pltpu.sync_copy(key_hbm.at[pl.ds(r * NB, NB)], key_v)
        pltpu.async_copy(table_hbm.at[idx_v.at[0]], rows_v, sem).wait()
        keys = key_v[...]
        pltpu.async_copy(edst_hbm.at[key_v], ed_v, sem).wait()
        lo4 = [jnp.zeros((NB,), jnp.float32) for _ in range(H)]
        for j in range(NB):
            ls = _dots(rows_v, ed_v, j, 0, None)
            jm = lane == j
            lo4 = [jnp.where(jm, ls[h], lo4[h]) for h in range(H)]
        valid = (r * NB + lane) < E
        for h in range(H):
            lo = lo4[h]
            lo = jnp.where(lo > 0, lo, NEG * lo)
            lo = jnp.where(valid, lo, BIG)
            lo_v[0, pl.ds(h * NB, NB)] = lo
            pm, ps = _prefix_ms(keys, lo)
            tix = keys * H + h
            mt = plsc.load_gather(mt_v, [tix])
            st = plsc.load_gather(st_v, [tix])
            mn = jnp.maximum(mt, pm)
            sn = st * jnp.exp(mt - mn) + ps * jnp.exp(pm - mn)
            re = _runend(keys)
            plsc.store_scatter(mt_v, [tix], mn, mask=re)
            plsc.store_scatter(st_v, [tix], sn, mask=re)
        pltpu.sync_copy(lo_v, lo_hbm.at[pl.ds(r, 1)])
        return 0

    lax.fori_loop(0, BPT, batch, 0)
    pltpu.sync_copy(mt_v, mt_hbm.at[w])
    pltpu.sync_copy(st_v, st_hbm.at[w])


def _sc_ainf(table, edst, idx2d, key2d):
    k = functools.partial(
        pl.kernel, mesh=_mesh(),
        compiler_params=pltpu.CompilerParams(needs_layout_passes=False),
        out_type=[jax.ShapeDtypeStruct((EPB, H * NB), jnp.float32),
                  jax.ShapeDtypeStruct((NW, TW), jnp.float32),
                  jax.ShapeDtypeStruct((NW, TW), jnp.float32)],
        scratch_types=[
            pltpu.VMEM((1, 2 * NB), jnp.int32),
            pltpu.VMEM((NB,), jnp.int32),
            pltpu.VMEM((2 * NB, D), jnp.float32),
            pltpu.VMEM((NB, H * D), jnp.float32),
            pltpu.VMEM((1, H * NB), jnp.float32),
            pltpu.VMEM((TW,), jnp.float32),
            pltpu.VMEM((TW,), jnp.float32),
            pltpu.SemaphoreType.DMA,
        ],
    )(_ainf_body)
    return k(table, edst, idx2d, key2d)


def _binf_body(table_hbm, idx_hbm, key_hbm, lo_hbm, mg_hbm, inv_hbm, out_hbm,
               idx_v, key_v, rows_v, lo_v, mt_v, iv_v, acc_v, stage_v,
               tgt_v, sem):
    w = _wid()
    sc = lax.axis_index("c")
    sid = lax.axis_index("s")
    lane = _lane()

    def zstage(i, _):
        for k in range(H * D // NB):
            stage_v[i, pl.ds(k * NB, NB)] = jnp.zeros((NB,), jnp.float32)
        return 0

    lax.fori_loop(0, NB, zstage, 0)

    def zrow(i, _):
        pltpu.sync_copy(
            stage_v,
            out_hbm.at[pl.ds(sc * PCH + sid * 640 + i * NB, NB)])
        return 0

    lax.fori_loop(0, 40, zrow, 0)
    pltpu.sync_copy(mg_hbm, mt_v)
    pltpu.sync_copy(inv_hbm, iv_v)

    def z2(i, _):
        acc_v[0, pl.ds(i * NB, NB)] = jnp.zeros((NB,), jnp.float32)
        return 0

    lax.fori_loop(0, H * D // NB, z2, 0)
    plsc.subcore_barrier()

    def batch(b, carry):
        prevkey = carry
        r = w * BPT + b
        pltpu.sync_copy(idx_hbm.at[pl.ds(r, 1)], idx_v)
        pltpu.sync_copy(key_hbm.at[pl.ds(r * NB, NB)], key_v)
        pltpu.async_copy(table_hbm.at[idx_v.at[0]], rows_v, sem).wait()
        pltpu.sync_copy(lo_hbm.at[pl.ds(r, 1)], lo_v)
        keys = key_v[...]
        att = []
        for h in range(H):
            lo = lo_v[0, pl.ds(h * NB, NB)]
            tix = keys * H + h
            m = plsc.load_gather(mt_v, [tix])
            iv = plsc.load_gather(iv_v, [tix])
            att.append(jnp.exp(lo - m) * iv)
        prev0 = _bcast(prevkey, NB - 1)
        for j in range(NB):
            kj = _bcast(keys, j)
            pj = kj == (prev0 if j == 0 else _bcast(keys, j - 1))
            zf = pj.astype(jnp.float32)
            aj = [_bcast(att[h], j) for h in range(H)]
            for k in range(D // NB):
                sl = pl.ds(k * NB, NB)
                u = rows_v[2 * j, sl] + rows_v[2 * j + 1, sl]
                for h in range(H):
                    osl = pl.ds(h * D + k * NB, NB)
                    nv = acc_v[0, osl] * zf + u * aj[h]
                    acc_v[0, osl] = nv
                    stage_v[j, osl] = nv
        re = _runend(keys)
        tgt_v[0, pl.ds(0, NB)] = (sc * PCH
                                  + jnp.where(re, keys, NPP + 128 + lane))
        pltpu.sync_copy(stage_v, out_hbm.at[tgt_v.at[0]])
        return keys

    lax.fori_loop(0, BPT, batch, jnp.full((NB,), -1, jnp.int32))


def _sc_binf(table, idx2d, key2d, lo, mg, inv):
    k = functools.partial(
        pl.kernel, mesh=_mesh(),
        compiler_params=pltpu.CompilerParams(needs_layout_passes=False),
        out_type=jax.ShapeDtypeStruct((2 * PCH, H * D), jnp.float32),
        scratch_types=[
            pltpu.VMEM((1, 2 * NB), jnp.int32),
            pltpu.VMEM((NB,), jnp.int32),
            pltpu.VMEM((2 * NB, D), jnp.float32),
            pltpu.VMEM((1, H * NB), jnp.float32),
            pltpu.VMEM((TW,), jnp.float32),
            pltpu.VMEM((TW,), jnp.float32),
            pltpu.VMEM((1, H * D), jnp.float32),
            pltpu.VMEM((NB, H * D), jnp.float32),
            pltpu.VMEM((1, NB), jnp.int32),
            pltpu.SemaphoreType.DMA,
        ],
    )(_binf_body)
    return k(table, idx2d, key2d, lo, mg, inv)


def _aout_body(table_hbm, esp_hbm, idx_hbm, key_hbm, typ_hbm,
               lo_hbm, mt_hbm, st_hbm,
               idx_v, key_v, typ_v, rows_v, es_v, lo_v, mt_v, st_v, sem):
    w = _wid()
    lane = _lane()

    def zinit(i, _):
        sl = pl.ds(i * NB, NB)
        mt_v[sl] = jnp.full((NB,), BIG, jnp.float32)
        st_v[sl] = jnp.zeros((NB,), jnp.float32)
        return 0

    lax.fori_loop(0, TW // NB, zinit, 0)

    def batch(b, _):
        r = w * BPT + b
        pltpu.sync_copy(idx_hbm.at[pl.ds(r, 1)], idx_v)
        pltpu.sync_copy(key_hbm.at[pl.ds(r * NB, NB)], key_v)
        pltpu.sync_copy(typ_hbm.at[pl.ds(r * NB, NB)], typ_v)
        pltpu.async_copy(table_hbm.at[idx_v.at[0]], rows_v, sem).wait()
        keys = key_v[...]
        pltpu.async_copy(esp_hbm.at[key_v], es_v, sem).wait()
        types = typ_v[...]
        lo4 = [jnp.zeros((NB,), jnp.float32) for _ in range(H)]
        for j in range(NB):
            rej = plsc.load_gather(
                es_v, [jnp.full((NB,), j, jnp.int32),
                       jnp.full((NB,), H * D, jnp.int32) + _bcast(types, j)])
            acc = [jnp.zeros((NB,), jnp.float32) for _ in range(H)]
            for k in range(D // NB):
                sl = pl.ds(k * NB, NB)
                ed = rows_v[j, sl]
                for h in range(H):
                    acc[h] = acc[h] + ed * es_v[j, pl.ds(h * D + k * NB, NB)]
            jm = lane == j
            for h in range(H):
                tot = _allsum(acc[h]) + rej
                lo4[h] = jnp.where(jm, tot, lo4[h])
        valid = (r * NB + lane) < E
        for h in range(H):
            lo = lo4[h]
            lo = jnp.where(lo > 0, lo, NEG * lo)
            lo = jnp.where(valid, lo, BIG)
            lo_v[0, pl.ds(h * NB, NB)] = lo
            pm, ps = _prefix_ms(keys, lo)
            tix = keys * H + h
            mt = plsc.load_gather(mt_v, [tix])
            st = plsc.load_gather(st_v, [tix])
            mn = jnp.maximum(mt, pm)
            sn = st * jnp.exp(mt - mn) + ps * jnp.exp(pm - mn)
            re = _runend(keys)
            plsc.store_scatter(mt_v, [tix], mn, mask=re)
            plsc.store_scatter(st_v, [tix], sn, mask=re)
        pltpu.sync_copy(lo_v, lo_hbm.at[pl.ds(r, 1)])
        return 0

    lax.fori_loop(0, BPT, batch, 0)
    pltpu.sync_copy(mt_v, mt_hbm.at[w])
    pltpu.sync_copy(st_v, st_hbm.at[w])


def _sc_aout(table, esp, idxo2d, key2d, typ2d):
    k = functools.partial(
        pl.kernel, mesh=_mesh(),
        compiler_params=pltpu.CompilerParams(needs_layout_passes=False),
        out_type=[jax.ShapeDtypeStruct((EPB, H * NB), jnp.float32),
                  jax.ShapeDtypeStruct((NW, TW), jnp.float32),
                  jax.ShapeDtypeStruct((NW, TW), jnp.float32)],
        scratch_types=[
            pltpu.VMEM((1, NB), jnp.int32),
            pltpu.VMEM((NB,), jnp.int32),
            pltpu.VMEM((NB,), jnp.int32),
            pltpu.VMEM((NB, D), jnp.float32),
            pltpu.VMEM((NB, ESP), jnp.float32),
            pltpu.VMEM((1, H * NB), jnp.float32),
            pltpu.VMEM((TW,), jnp.float32),
            pltpu.VMEM((TW,), jnp.float32),
            pltpu.SemaphoreType.DMA,
        ],
    )(_aout_body)
    return k(table, esp, idxo2d, key2d, typ2d)


def _bout_body(lo_hbm, key_hbm, dst_hbm, a_hbm, mg_hbm, inv_hbm, ap_hbm,
               key_v, dst_v, lo_v, mt_v, iv_v, a_v, acc_v, sem):
    w = _wid()
    lane = _lane()
    pltpu.sync_copy(mg_hbm, mt_v)
    pltpu.sync_copy(inv_hbm, iv_v)
    pltpu.sync_copy(a_hbm, a_v)

    def z(i, _):
        acc_v[pl.ds(i * NB, NB)] = jnp.zeros((NB,), jnp.float32)
        return 0

    lax.fori_loop(0, ACCW // NB, z, 0)

    def batch(b, _):
        r = w * BPT + b
        pltpu.sync_copy(key_hbm.at[pl.ds(r * NB, NB)], key_v)
        pltpu.sync_copy(dst_hbm.at[pl.ds(r * NB, NB)], dst_v)
        pltpu.sync_copy(lo_hbm.at[pl.ds(r, 1)], lo_v)
        keys = key_v[...]
        dsts = dst_v[...]
        tp = jnp.zeros((NB,), jnp.float32)
        for h in range(H):
            lo = lo_v[0, pl.ds(h * NB, NB)]
            tix = keys * H + h
            m = plsc.load_gather(mt_v, [tix])
            iv = plsc.load_gather(iv_v, [tix])
            tp = tp + jnp.exp(lo - m) * iv
        asrc = plsc.load_gather(a_v, [keys])
        valid = (r * NB + lane) < E
        c = jnp.where(valid, tp * asrc * (1.0 / H), 0.0)
        plsc.addupdate_scatter(acc_v, [dsts], c, mask=valid)
        return 0

    lax.fori_loop(0, BPT, batch, 0)
    pltpu.sync_copy(acc_v, ap_hbm.at[w])


def _sc_bout(lo, key2d, dst2d, a_in, mg, inv):
    k = functools.partial(
        pl.kernel, mesh=_mesh(),
        compiler_params=pltpu.CompilerParams(needs_layout_passes=False),
        out_type=jax.ShapeDtypeStruct((NW, ACCW), jnp.float32),
        scratch_types=[
            pltpu.VMEM((NB,), jnp.int32),
            pltpu.VMEM((NB,), jnp.int32),
            pltpu.VMEM((1, H * NB), jnp.float32),
            pltpu.VMEM((TW,), jnp.float32),
            pltpu.VMEM((TW,), jnp.float32),
            pltpu.VMEM((ACCW,), jnp.float32),
            pltpu.VMEM((ACCW,), jnp.float32),
            pltpu.SemaphoreType.DMA,
        ],
    )(_bout_body)
    return k(lo, key2d, dst2d, a_in, mg, inv)


def kernel(entity_emb, relation_emb, fc_w, w_q, w_h_entity, w_h_dialogue,
           out_w_init, out_w_q, dialogue_context, node_ids, edge_types,
           edge_index, seed_set):
    i32 = jnp.int32
    src = edge_index[0]
    dst = edge_index[1]
    seeds = seed_set.astype(i32)

    perm_d = jnp.argsort(dst)
    srcd = src[perm_d].astype(i32)
    td = edge_types[perm_d].astype(i32)
    dstd = dst[perm_d].astype(i32)
    perm_s = jnp.argsort(src)
    dsts = dst[perm_s].astype(i32)
    ts = edge_types[perm_s].astype(i32)
    srcs = src[perm_s].astype(i32)

    pad = EP - E
    idx_in = jnp.stack([srcd, N + td], 1).reshape(-1)
    idx_in = jnp.pad(idx_in, (0, 2 * pad)).reshape(EPB, 2 * NB)
    dstd_p = jnp.pad(dstd, (0, pad), constant_values=N)
    srcs_p = jnp.pad(srcs, (0, pad), constant_values=N)
    dsts_p = jnp.pad(dsts, (0, pad), constant_values=N)
    ts_p = jnp.pad(ts, (0, pad))
    idx_out = jnp.pad(dsts, (0, pad)).reshape(EPB, NB)

    X = jnp.concatenate([entity_emb, relation_emb], 0)
    Xp = jnp.pad(X, ((0, TPAD - NT), (0, 0)))
    table0 = _tc_project(Xp, fc_w)
    suffix = table0[N:]
    frel_pad = jnp.pad(table0[N:NT], ((0, 128 - NREL), (0, 0)))
    dctx = dialogue_context

    def inflow(table_i, edst_i):
        lo, mtab, stab = _sc_ainf(table_i, edst_i, idx_in, dstd_p)
        mg, inv = _tc_merge(mtab, stab)
        part = _sc_binf(table_i, idx_in, dstd_p, lo, mg.reshape(-1),
                        inv.reshape(-1))
        return part.reshape(2, PCH, H * D)

    def outflow(table_i, esp_i, a_in):
        lo, mtab, stab = _sc_aout(table_i, esp_i, idx_out, srcs_p, ts_p)
        mg, inv = _tc_merge(mtab, stab)
        ap = _sc_bout(lo, srcs_p, dsts_p, a_in, mg.reshape(-1),
                      inv.reshape(-1))
        return _tc_combine(ap).reshape(-1)

    edst0 = _tc_edst(table0[:N + NB], w_q, N + NB)
    part = inflow(table0, edst0)
    ef1, edst1 = _tc_iter(part, w_h_entity, dctx, w_h_dialogue, w_q, True)
    table1 = jnp.concatenate([ef1[:N], suffix], 0)
    part = inflow(table1, edst1)
    ef2, edst2 = _tc_iter(part, w_h_entity, dctx, w_h_dialogue, w_q, True)
    table2 = jnp.concatenate([ef2[:N], suffix], 0)
    part = inflow(table2, edst2)
    (ef3,) = _tc_iter(part, w_h_entity, dctx, w_h_dialogue, w_q, False)
    table3 = jnp.concatenate([ef3[:N], suffix], 0)

    a0 = _tc_csa(ef1[:N], dctx, out_w_init, seeds)
    a0p = jnp.pad(a0.reshape(-1), (0, ACCW - N))

    esp1 = _tc_esrcp(table2[:N + NB], out_w_q, frel_pad)
    a1 = outflow(table2, esp1, a0p)
    esp2 = _tc_esrcp(table3[:N + NB], out_w_q, frel_pad)
    a2 = outflow(table3, esp2, a1)
    return a2[:N]


# speedup vs baseline: 4.7342x; 1.5060x over previous
import functools

import jax
import jax.numpy as jnp
from jax import lax
from jax.experimental import pallas as pl
from jax.experimental.pallas import tpu as pltpu
from jax.experimental.pallas import tpu_sc as plsc

N = 10000
E = 160000
D = 256
H = 4
NREL = 50
NEG = 0.01
NT = N + NREL
TPAD = 10240
NW = 32
NB = 16
EPB = 10112
EP = EPB * NB
BPT = EPB // NW
NPP = N + 48
TW = NPP * H
ACCW = 10112
PCH = NPP + 192
ESP = H * D + 128
BIG = -1e30


def _mm(a, b, ca, cb):
    return lax.dot_general(a, b, (((ca,), (cb,)), ((), ())),
                           preferred_element_type=jnp.float32)


def _proj_body(x_ref, w_ref, o_ref):
    o_ref[...] = _mm(x_ref[...], w_ref[...], 1, 1)


def _tc_project(x, w):
    R = x.shape[0]
    return pl.pallas_call(
        _proj_body,
        grid=(R // 1024,),
        in_specs=[pl.BlockSpec((1024, D), lambda i: (i, 0)),
                  pl.BlockSpec((D, D), lambda i: (0, 0))],
        out_specs=pl.BlockSpec((1024, D), lambda i: (i, 0)),
        out_shape=jax.ShapeDtypeStruct((R, D), jnp.float32),
    )(x, w)


def _edst_body(x_ref, wq_ref, o_ref):
    x = x_ref[...]
    for h in range(H):
        o_ref[:, h * D:(h + 1) * D] = _mm(x, wq_ref[h], 1, 0)


def _tc_edst(ent, wq, rows_out):
    return pl.pallas_call(
        _edst_body,
        grid=(10,),
        in_specs=[pl.BlockSpec((1024, D), lambda i: (i, 0)),
                  pl.BlockSpec((H, D, D), lambda i: (0, 0, 0))],
        out_specs=pl.BlockSpec((1024, H * D), lambda i: (i, 0)),
        out_shape=jax.ShapeDtypeStruct((rows_out, H * D), jnp.float32),
    )(ent, wq)


def _esrcp_body(x_ref, wq_ref, fr_ref, o_ref):
    x = x_ref[...]
    for h in range(H):
        o_ref[:, h * D:(h + 1) * D] = _mm(x, wq_ref[h], 1, 0)
    o_ref[:, H * D:] = _mm(x, fr_ref[...], 1, 1)


def _tc_esrcp(ent, wq, frel_pad):
    return pl.pallas_call(
        _esrcp_body,
        grid=(10,),
        in_specs=[pl.BlockSpec((1024, D), lambda i: (i, 0)),
                  pl.BlockSpec((H, D, D), lambda i: (0, 0, 0)),
                  pl.BlockSpec((128, D), lambda i: (0, 0))],
        out_specs=pl.BlockSpec((1024, ESP), lambda i: (i, 0)),
        out_shape=jax.ShapeDtypeStruct((N + NB, ESP), jnp.float32),
    )(ent, wq, frel_pad)


def _merge_body(m_ref, s_ref, mg_ref, inv_ref):
    m = m_ref[...]
    s = s_ref[...]
    mg = jnp.max(m, axis=0, keepdims=True)
    sg = jnp.sum(s * jnp.exp(m - mg), axis=0, keepdims=True)
    mg_ref[...] = mg
    inv_ref[...] = 1.0 / (sg + 1e-16)


def _tc_merge(mtab, stab):
    return pl.pallas_call(
        _merge_body,
        in_specs=[pl.BlockSpec((NW, TW), lambda: (0, 0)),
                  pl.BlockSpec((NW, TW), lambda: (0, 0))],
        out_specs=[pl.BlockSpec((1, TW), lambda: (0, 0)),
                   pl.BlockSpec((1, TW), lambda: (0, 0))],
        out_shape=[jax.ShapeDtypeStruct((1, TW), jnp.float32),
                   jax.ShapeDtypeStruct((1, TW), jnp.float32)],
    )(mtab, stab)


def _iter_body(p_ref, wh_ref, dctx_ref, whd_ref, wq_ref, ef_ref, edst_ref):
    agg = p_ref[0] + p_ref[1]
    dch = _mm(dctx_ref[...], whd_ref[...], 1, 0)
    ef = _mm(agg, wh_ref[...], 1, 0) + dch
    ef_ref[...] = ef
    if edst_ref is not None:
        for h in range(H):
            edst_ref[:, h * D:(h + 1) * D] = _mm(ef, wq_ref[h], 1, 0)


def _tc_iter(partial, wh, dctx, whd, wq, with_q):
    if with_q:
        body = _iter_body
        out_specs = [pl.BlockSpec((1024, D), lambda i: (i, 0)),
                     pl.BlockSpec((1024, H * D), lambda i: (i, 0))]
        out_shape = [jax.ShapeDtypeStruct((N + NB, D), jnp.float32),
                     jax.ShapeDtypeStruct((N + NB, H * D), jnp.float32)]
    else:
        def body(a, b, c, d, e, f):
            return _iter_body(a, b, c, d, e, f, None)
        out_specs = [pl.BlockSpec((1024, D), lambda i: (i, 0))]
        out_shape = [jax.ShapeDtypeStruct((N + NB, D), jnp.float32)]
    return pl.pallas_call(
        body,
        grid=(10,),
        in_specs=[
            pl.BlockSpec((2, 1024, H * D), lambda i: (0, i, 0)),
            pl.BlockSpec((H * D, D), lambda i: (0, 0)),
            pl.BlockSpec((1, D), lambda i: (0, 0)),
            pl.BlockSpec((D, D), lambda i: (0, 0)),
            pl.BlockSpec((H, D, D), lambda i: (0, 0, 0)),
        ],
        out_specs=out_specs,
        out_shape=out_shape,
    )(partial, wh, dctx, whd, wq)


def _csa_body(ef_ref, dctx_ref, owi_ref, seeds_ref, o_ref):
    dcv = _mm(dctx_ref[...], owi_ref[...], 1, 0)
    csa = _mm(dcv, ef_ref[...], 1, 1)
    ids = lax.broadcasted_iota(jnp.int32, (1, N), 1)
    for i in range(32):
        csa = csa + jnp.where(ids == seeds_ref[i], 10000.0, 0.0)
    csa = csa - 10000.0
    m = jnp.max(csa)
    ex = jnp.exp(csa - m)
    o_ref[...] = ex / jnp.sum(ex)


def _tc_csa(ef, dctx, owi, seeds):
    return pl.pallas_call(
        _csa_body,
        in_specs=[
            pl.BlockSpec((N, D), lambda: (0, 0)),
            pl.BlockSpec((1, D), lambda: (0, 0)),
            pl.BlockSpec((D, D), lambda: (0, 0)),
            pl.BlockSpec(memory_space=pltpu.SMEM),
        ],
        out_specs=pl.BlockSpec((1, N), lambda: (0, 0)),
        out_shape=jax.ShapeDtypeStruct((1, N), jnp.float32),
    )(ef, dctx, owi, seeds)


def _comb_body(ap_ref, o_ref):
    o_ref[...] = jnp.sum(ap_ref[...], axis=0, keepdims=True)


def _tc_combine(apart):
    return pl.pallas_call(
        _comb_body,
        in_specs=[pl.BlockSpec((NW, ACCW), lambda: (0, 0))],
        out_specs=pl.BlockSpec((1, ACCW), lambda: (0, 0)),
        out_shape=jax.ShapeDtypeStruct((1, ACCW), jnp.float32),
    )(apart)


def _mesh():
    return plsc.VectorSubcoreMesh(core_axis_name="c", subcore_axis_name="s")


def _lane():
    return lax.broadcasted_iota(jnp.int32, (NB,), 0)


def _bcast(v, j):
    return v[jnp.full((NB,), j, jnp.int32)]


def _allsum(v):
    lane = _lane()
    for sh in (8, 4, 2, 1):
        v = v + v[lane ^ sh]
    return v


def _prefix_ms(keys, lo):
    lane = _lane()
    m = lo
    s = jnp.ones((NB,), jnp.float32)
    for sh in (1, 2, 4, 8):
        idx = jnp.maximum(lane - sh, 0)
        pm = m[idx]
        ps = s[idx]
        same = (keys[idx] == keys) & (lane >= sh)
        mn = jnp.maximum(m, jnp.where(same, pm, BIG))
        s = s * jnp.exp(m - mn) + jnp.where(same, ps * jnp.exp(pm - mn), 0.0)
        m = mn
    return m, s


def _runend(keys):
    lane = _lane()
    nxt = keys[jnp.minimum(lane + 1, NB - 1)]
    return (keys != nxt) | (lane == NB - 1)


def _wid():
    return lax.axis_index("s") * 2 + lax.axis_index("c")


def _dots(rows_v, q_v, j, qoff, extra):
    acc = [jnp.zeros((NB,), jnp.float32) for _ in range(H)]
    for k in range(D // NB):
        sl = pl.ds(k * NB, NB)
        u = rows_v[2 * j, sl] + rows_v[2 * j + 1, sl]
        for h in range(H):
            acc[h] = acc[h] + u * q_v[j, pl.ds(qoff + h * D + k * NB, NB)]
    return [_allsum(a) + (0.0 if extra is None else extra) for a in acc]


def _ainf_body(table_hbm, edst_hbm, idx_hbm, key_hbm, lo_hbm, mt_hbm, st_hbm,
               idx_v, key_v, rows_v, ed_v, lo_v, mt_v, st_v,
               sr0, sr1, se0, se1):
    w = _wid()
    lane = _lane()

    def zinit(i, _):
        sl = pl.ds(i * NB, NB)
        mt_v[sl] = jnp.full((NB,), BIG, jnp.float32)
        st_v[sl] = jnp.zeros((NB,), jnp.float32)
        return 0

    lax.fori_loop(0, TW // NB, zinit, 0)
    srows = (sr0, sr1)

    def fetch_idx(b, q):
        r = w * BPT + b
        pltpu.sync_copy(idx_hbm.at[pl.ds(r, 1)], idx_v.at[pl.ds(q, 1)])
        pltpu.sync_copy(key_hbm.at[pl.ds(r * NB, NB)], key_v.at[q])

    def start_rows(q):
        pltpu.make_async_copy(table_hbm.at[idx_v.at[q]],
                              rows_v.at[q], srows[q]).start()

    def wait_rows(q):
        pltpu.make_async_copy(table_hbm.at[idx_v.at[q]],
                              rows_v.at[q], srows[q]).wait()

    def start_ed(q):
        pltpu.make_async_copy(edst_hbm.at[key_v.at[q]], ed_v, se0).start()

    def wait_ed(q):
        pltpu.make_async_copy(edst_hbm.at[key_v.at[q]], ed_v, se0).wait()

    fetch_idx(0, 0)
    start_rows(0)
    start_ed(0)
    fetch_idx(1, 1)

    def one(b, q):
        r = w * BPT + b

        @pl.when(b + 1 < BPT)
        def _pre():
            start_rows(1 - q)

        wait_rows(q)
        wait_ed(q)
        keys = key_v[q, :]

        def edge(j, lo4):
            ls = _dots(rows_v.at[q], ed_v, j, 0, None)
            jm = lane == j
            return tuple(jnp.where(jm, ls[h], lo4[h]) for h in range(H))

        lo4 = lax.fori_loop(
            0, NB, edge,
            tuple(jnp.zeros((NB,), jnp.float32) for _ in range(H)))

        @pl.when(b + 1 < BPT)
        def _pree():
            start_ed(1 - q)

        valid = (r * NB + lane) < E
        for h in range(H):
            lo = lo4[h]
            lo = jnp.where(lo > 0, lo, NEG * lo)
            lo = jnp.where(valid, lo, BIG)
            lo_v[0, pl.ds(h * NB, NB)] = lo
            pm, ps = _prefix_ms(keys, lo)
            tix = keys * H + h
            mt = plsc.load_gather(mt_v, [tix])
            st = plsc.load_gather(st_v, [tix])
            mn = jnp.maximum(mt, pm)
            sn = st * jnp.exp(mt - mn) + ps * jnp.exp(pm - mn)
            re = _runend(keys)
            plsc.store_scatter(mt_v, [tix], mn, mask=re)
            plsc.store_scatter(st_v, [tix], sn, mask=re)
        pltpu.sync_copy(lo_v, lo_hbm.at[pl.ds(r, 1)])

        @pl.when(b + 2 < BPT)
        def _nidx():
            fetch_idx(b + 2, q)

    def pair(bb, _):
        one(2 * bb, 0)
        one(2 * bb + 1, 1)
        return 0

    lax.fori_loop(0, BPT // 2, pair, 0)
    pltpu.sync_copy(mt_v, mt_hbm.at[w])
    pltpu.sync_copy(st_v, st_hbm.at[w])


def _sc_ainf(table, edst, idx2d, key2d):
    k = functools.partial(
        pl.kernel, mesh=_mesh(),
        compiler_params=pltpu.CompilerParams(needs_layout_passes=False),
        out_type=[jax.ShapeDtypeStruct((EPB, H * NB), jnp.float32),
                  jax.ShapeDtypeStruct((NW, TW), jnp.float32),
                  jax.ShapeDtypeStruct((NW, TW), jnp.float32)],
        scratch_types=[
            pltpu.VMEM((2, 2 * NB), jnp.int32),
            pltpu.VMEM((2, NB), jnp.int32),
            pltpu.VMEM((2, 2 * NB, D), jnp.float32),
            pltpu.VMEM((NB, H * D), jnp.float32),
            pltpu.VMEM((1, H * NB), jnp.float32),
            pltpu.VMEM((TW,), jnp.float32),
            pltpu.VMEM((TW,), jnp.float32),
            pltpu.SemaphoreType.DMA,
            pltpu.SemaphoreType.DMA,
            pltpu.SemaphoreType.DMA,
            pltpu.SemaphoreType.DMA,
        ],
    )(_ainf_body)
    return k(table, edst, idx2d, key2d)


def _binf_body(table_hbm, idx_hbm, key_hbm, lo_hbm, mg_hbm, inv_hbm, out_hbm,
               idx_v, key_v, rows_v, lo_v, mt_v, iv_v, acc_v, stage_v,
               tgt_v, sem):
    w = _wid()
    sc = lax.axis_index("c")
    sid = lax.axis_index("s")
    lane = _lane()

    def zstage(i, _):
        for k in range(H * D // NB):
            stage_v[i, pl.ds(k * NB, NB)] = jnp.zeros((NB,), jnp.float32)
        return 0

    lax.fori_loop(0, NB, zstage, 0)

    def zrow(i, _):
        pltpu.sync_copy(
            stage_v,
            out_hbm.at[pl.ds(sc * PCH + sid * 640 + i * NB, NB)])
        return 0

    lax.fori_loop(0, 40, zrow, 0)
    pltpu.sync_copy(mg_hbm, mt_v)
    pltpu.sync_copy(inv_hbm, iv_v)

    def z2(i, _):
        acc_v[0, pl.ds(i * NB, NB)] = jnp.zeros((NB,), jnp.float32)
        return 0

    lax.fori_loop(0, H * D // NB, z2, 0)
    plsc.subcore_barrier()

    def batch(b, carry):
        prevkey = carry
        r = w * BPT + b
        pltpu.sync_copy(idx_hbm.at[pl.ds(r, 1)], idx_v)
        pltpu.sync_copy(key_hbm.at[pl.ds(r * NB, NB)], key_v)
        pltpu.async_copy(table_hbm.at[idx_v.at[0]], rows_v, sem).wait()
        pltpu.sync_copy(lo_hbm.at[pl.ds(r, 1)], lo_v)
        keys = key_v[...]
        att = []
        for h in range(H):
            lo = lo_v[0, pl.ds(h * NB, NB)]
            tix = keys * H + h
            m = plsc.load_gather(mt_v, [tix])
            iv = plsc.load_gather(iv_v, [tix])
            att.append(jnp.exp(lo - m) * iv)
        prev0 = _bcast(prevkey, NB - 1)

        def edge(j, _):
            kj = keys[jnp.full((NB,), j, jnp.int32)]
            km = keys[jnp.maximum(jnp.full((NB,), j, jnp.int32) - 1, 0)]
            isz = jnp.full((NB,), j, jnp.int32) == 0
            pj = kj == jnp.where(isz, prev0, km)
            zf = pj.astype(jnp.float32)
            aj = [att[h][jnp.full((NB,), j, jnp.int32)] for h in range(H)]
            for k in range(D // NB):
                sl = pl.ds(k * NB, NB)
                u = rows_v[2 * j, sl] + rows_v[2 * j + 1, sl]
                for h in range(H):
                    osl = pl.ds(h * D + k * NB, NB)
                    nv = acc_v[0, osl] * zf + u * aj[h]
                    acc_v[0, osl] = nv
                    stage_v[j, osl] = nv
            return 0

        lax.fori_loop(0, NB, edge, 0)
        re = _runend(keys)
        tgt_v[0, pl.ds(0, NB)] = (sc * PCH
                                  + jnp.where(re, keys, NPP + 128 + lane))
        pltpu.sync_copy(stage_v, out_hbm.at[tgt_v.at[0]])
        return keys

    lax.fori_loop(0, BPT, batch, jnp.full((NB,), -1, jnp.int32))


def _sc_binf(table, idx2d, key2d, lo, mg, inv):
    k = functools.partial(
        pl.kernel, mesh=_mesh(),
        compiler_params=pltpu.CompilerParams(needs_layout_passes=False),
        out_type=jax.ShapeDtypeStruct((2 * PCH, H * D), jnp.float32),
        scratch_types=[
            pltpu.VMEM((1, 2 * NB), jnp.int32),
            pltpu.VMEM((NB,), jnp.int32),
            pltpu.VMEM((2 * NB, D), jnp.float32),
            pltpu.VMEM((1, H * NB), jnp.float32),
            pltpu.VMEM((TW,), jnp.float32),
            pltpu.VMEM((TW,), jnp.float32),
            pltpu.VMEM((1, H * D), jnp.float32),
            pltpu.VMEM((NB, H * D), jnp.float32),
            pltpu.VMEM((1, NB), jnp.int32),
            pltpu.SemaphoreType.DMA,
        ],
    )(_binf_body)
    return k(table, idx2d, key2d, lo, mg, inv)


def _aout_body(table_hbm, esp_hbm, idx_hbm, key_hbm, typ_hbm,
               lo_hbm, mt_hbm, st_hbm,
               idx_v, key_v, typ_v, rows_v, es_v, lo_v, mt_v, st_v,
               sr0, sr1, se0, se1):
    w = _wid()
    lane = _lane()

    def zinit(i, _):
        sl = pl.ds(i * NB, NB)
        mt_v[sl] = jnp.full((NB,), BIG, jnp.float32)
        st_v[sl] = jnp.zeros((NB,), jnp.float32)
        return 0

    lax.fori_loop(0, TW // NB, zinit, 0)
    srows = (sr0, sr1)

    def fetch_idx(b, q):
        r = w * BPT + b
        pltpu.sync_copy(idx_hbm.at[pl.ds(r, 1)], idx_v.at[pl.ds(q, 1)])
        pltpu.sync_copy(key_hbm.at[pl.ds(r * NB, NB)], key_v.at[q])
        pltpu.sync_copy(typ_hbm.at[pl.ds(r * NB, NB)], typ_v.at[q])

    def start_rows(q):
        pltpu.make_async_copy(table_hbm.at[idx_v.at[q]],
                              rows_v.at[q], srows[q]).start()

    def wait_rows(q):
        pltpu.make_async_copy(table_hbm.at[idx_v.at[q]],
                              rows_v.at[q], srows[q]).wait()

    def start_es(q):
        pltpu.make_async_copy(esp_hbm.at[key_v.at[q]], es_v, se0).start()

    def wait_es(q):
        pltpu.make_async_copy(esp_hbm.at[key_v.at[q]], es_v, se0).wait()

    fetch_idx(0, 0)
    start_rows(0)
    start_es(0)
    fetch_idx(1, 1)

    def one(b, q):
        r = w * BPT + b

        @pl.when(b + 1 < BPT)
        def _pre():
            start_rows(1 - q)

        wait_rows(q)
        wait_es(q)
        keys = key_v[q, :]
        types = typ_v[q, :]
        esq = es_v
        rowsq = rows_v.at[q]

        def edge(j, lo4):
            rej = plsc.load_gather(
                esq, [jnp.full((NB,), j, jnp.int32),
                      jnp.full((NB,), H * D, jnp.int32) + _bcast(types, j)])
            acc = [jnp.zeros((NB,), jnp.float32) for _ in range(H)]
            for k in range(D // NB):
                sl = pl.ds(k * NB, NB)
                ed = rowsq[j, sl]
                for h in range(H):
                    acc[h] = acc[h] + ed * esq[j, pl.ds(h * D + k * NB, NB)]
            jm = lane == j
            return tuple(jnp.where(jm, _allsum(acc[h]) + rej, lo4[h])
                         for h in range(H))

        lo4 = lax.fori_loop(
            0, NB, edge,
            tuple(jnp.zeros((NB,), jnp.float32) for _ in range(H)))

        @pl.when(b + 1 < BPT)
        def _prees():
            start_es(1 - q)

        valid = (r * NB + lane) < E
        for h in range(H):
            lo = lo4[h]
            lo = jnp.where(lo > 0, lo, NEG * lo)
            lo = jnp.where(valid, lo, BIG)
            lo_v[0, pl.ds(h * NB, NB)] = lo
            pm, ps = _prefix_ms(keys, lo)
            tix = keys * H + h
            mt = plsc.load_gather(mt_v, [tix])
            st = plsc.load_gather(st_v, [tix])
            mn = jnp.maximum(mt, pm)
            sn = st * jnp.exp(mt - mn) + ps * jnp.exp(pm - mn)
            re = _runend(keys)
            plsc.store_scatter(mt_v, [tix], mn, mask=re)
            plsc.store_scatter(st_v, [tix], sn, mask=re)
        pltpu.sync_copy(lo_v, lo_hbm.at[pl.ds(r, 1)])

        @pl.when(b + 2 < BPT)
        def _nidx():
            fetch_idx(b + 2, q)

    def pair(bb, _):
        one(2 * bb, 0)
        one(2 * bb + 1, 1)
        return 0

    lax.fori_loop(0, BPT // 2, pair, 0)
    pltpu.sync_copy(mt_v, mt_hbm.at[w])
    pltpu.sync_copy(st_v, st_hbm.at[w])


def _sc_aout(table, esp, idxo2d, key2d, typ2d):
    k = functools.partial(
        pl.kernel, mesh=_mesh(),
        compiler_params=pltpu.CompilerParams(needs_layout_passes=False),
        out_type=[jax.ShapeDtypeStruct((EPB, H * NB), jnp.float32),
                  jax.ShapeDtypeStruct((NW, TW), jnp.float32),
                  jax.ShapeDtypeStruct((NW, TW), jnp.float32)],
        scratch_types=[
            pltpu.VMEM((2, NB), jnp.int32),
            pltpu.VMEM((2, NB), jnp.int32),
            pltpu.VMEM((2, NB), jnp.int32),
            pltpu.VMEM((2, NB, D), jnp.float32),
            pltpu.VMEM((NB, ESP), jnp.float32),
            pltpu.VMEM((1, H * NB), jnp.float32),
            pltpu.VMEM((TW,), jnp.float32),
            pltpu.VMEM((TW,), jnp.float32),
            pltpu.SemaphoreType.DMA,
            pltpu.SemaphoreType.DMA,
            pltpu.SemaphoreType.DMA,
            pltpu.SemaphoreType.DMA,
        ],
    )(_aout_body)
    return k(table, esp, idxo2d, key2d, typ2d)


def _bout_body(lo_hbm, key_hbm, dst_hbm, a_hbm, mg_hbm, inv_hbm, ap_hbm,
               key_v, dst_v, lo_v, mt_v, iv_v, a_v, acc_v, sem):
    w = _wid()
    lane = _lane()
    pltpu.sync_copy(mg_hbm, mt_v)
    pltpu.sync_copy(inv_hbm, iv_v)
    pltpu.sync_copy(a_hbm, a_v)

    def z(i, _):
        acc_v[pl.ds(i * NB, NB)] = jnp.zeros((NB,), jnp.float32)
        return 0

    lax.fori_loop(0, ACCW // NB, z, 0)

    def batch(b, _):
        r = w * BPT + b
        pltpu.sync_copy(key_hbm.at[pl.ds(r * NB, NB)], key_v)
        pltpu.sync_copy(dst_hbm.at[pl.ds(r * NB, NB)], dst_v)
        pltpu.sync_copy(lo_hbm.at[pl.ds(r, 1)], lo_v)
        keys = key_v[...]
        dsts = dst_v[...]
        tp = jnp.zeros((NB,), jnp.float32)
        for h in range(H):
            lo = lo_v[0, pl.ds(h * NB, NB)]
            tix = keys * H + h
            m = plsc.load_gather(mt_v, [tix])
            iv = plsc.load_gather(iv_v, [tix])
            tp = tp + jnp.exp(lo - m) * iv
        asrc = plsc.load_gather(a_v, [keys])
        valid = (r * NB + lane) < E
        c = jnp.where(valid, tp * asrc * (1.0 / H), 0.0)
        plsc.addupdate_scatter(acc_v, [dsts], c, mask=valid)
        return 0

    lax.fori_loop(0, BPT, batch, 0)
    pltpu.sync_copy(acc_v, ap_hbm.at[w])


def _sc_bout(lo, key2d, dst2d, a_in, mg, inv):
    k = functools.partial(
        pl.kernel, mesh=_mesh(),
        compiler_params=pltpu.CompilerParams(needs_layout_passes=False),
        out_type=jax.ShapeDtypeStruct((NW, ACCW), jnp.float32),
        scratch_types=[
            pltpu.VMEM((NB,), jnp.int32),
            pltpu.VMEM((NB,), jnp.int32),
            pltpu.VMEM((1, H * NB), jnp.float32),
            pltpu.VMEM((TW,), jnp.float32),
            pltpu.VMEM((TW,), jnp.float32),
            pltpu.VMEM((ACCW,), jnp.float32),
            pltpu.VMEM((ACCW,), jnp.float32),
            pltpu.SemaphoreType.DMA,
        ],
    )(_bout_body)
    return k(lo, key2d, dst2d, a_in, mg, inv)


def kernel(entity_emb, relation_emb, fc_w, w_q, w_h_entity, w_h_dialogue,
           out_w_init, out_w_q, dialogue_context, node_ids, edge_types,
           edge_index, seed_set):
    i32 = jnp.int32
    src = edge_index[0]
    dst = edge_index[1]
    seeds = seed_set.astype(i32)

    perm_d = jnp.argsort(dst)
    srcd = src[perm_d].astype(i32)
    td = edge_types[perm_d].astype(i32)
    dstd = dst[perm_d].astype(i32)
    perm_s = jnp.argsort(src)
    dsts = dst[perm_s].astype(i32)
    ts = edge_types[perm_s].astype(i32)
    srcs = src[perm_s].astype(i32)

    pad = EP - E
    idx_in = jnp.stack([srcd, N + td], 1).reshape(-1)
    idx_in = jnp.pad(idx_in, (0, 2 * pad)).reshape(EPB, 2 * NB)
    dstd_p = jnp.pad(dstd, (0, pad), constant_values=N)
    srcs_p = jnp.pad(srcs, (0, pad), constant_values=N)
    dsts_p = jnp.pad(dsts, (0, pad), constant_values=N)
    ts_p = jnp.pad(ts, (0, pad))
    idx_out = jnp.pad(dsts, (0, pad)).reshape(EPB, NB)

    X = jnp.concatenate([entity_emb, relation_emb], 0)
    Xp = jnp.pad(X, ((0, TPAD - NT), (0, 0)))
    table0 = _tc_project(Xp, fc_w)
    suffix = table0[N:]
    frel_pad = jnp.pad(table0[N:NT], ((0, 128 - NREL), (0, 0)))
    dctx = dialogue_context

    def inflow(table_i, edst_i):
        lo, mtab, stab = _sc_ainf(table_i, edst_i, idx_in, dstd_p)
        mg, inv = _tc_merge(mtab, stab)
        part = _sc_binf(table_i, idx_in, dstd_p, lo, mg.reshape(-1),
                        inv.reshape(-1))
        return part.reshape(2, PCH, H * D)

    def outflow(table_i, esp_i, a_in):
        lo, mtab, stab = _sc_aout(table_i, esp_i, idx_out, srcs_p, ts_p)
        mg, inv = _tc_merge(mtab, stab)
        ap = _sc_bout(lo, srcs_p, dsts_p, a_in, mg.reshape(-1),
                      inv.reshape(-1))
        return _tc_combine(ap).reshape(-1)

    edst0 = _tc_edst(table0[:N + NB], w_q, N + NB)
    part = inflow(table0, edst0)
    ef1, edst1 = _tc_iter(part, w_h_entity, dctx, w_h_dialogue, w_q, True)
    table1 = jnp.concatenate([ef1[:N], suffix], 0)
    part = inflow(table1, edst1)
    ef2, edst2 = _tc_iter(part, w_h_entity, dctx, w_h_dialogue, w_q, True)
    table2 = jnp.concatenate([ef2[:N], suffix], 0)
    part = inflow(table2, edst2)
    (ef3,) = _tc_iter(part, w_h_entity, dctx, w_h_dialogue, w_q, False)
    table3 = jnp.concatenate([ef3[:N], suffix], 0)

    a0 = _tc_csa(ef1[:N], dctx, out_w_init, seeds)
    a0p = jnp.pad(a0.reshape(-1), (0, ACCW - N))

    esp1 = _tc_esrcp(table2[:N + NB], out_w_q, frel_pad)
    a1 = outflow(table2, esp1, a0p)
    esp2 = _tc_esrcp(table3[:N + NB], out_w_q, frel_pad)
    a2 = outflow(table3, esp2, a1)
    return a2[:N]


# speedup vs baseline: 5.1845x; 1.0951x over previous
import functools

import jax
import jax.numpy as jnp
from jax import lax
from jax.experimental import pallas as pl
from jax.experimental.pallas import tpu as pltpu
from jax.experimental.pallas import tpu_sc as plsc

N = 10000
E = 160000
D = 256
H = 4
NREL = 50
NEG = 0.01
NT = N + NREL
TPAD = 10240
NW = 32
NB = 16
EPB = 10112
EP = EPB * NB
BPT = EPB // NW
NPP = N + 48
TW = NPP * H
ACCW = 10112
PCH = NPP + 192
ESP = H * D + 128
BIG = -1e30


def _mm(a, b, ca, cb):
    return lax.dot_general(a, b, (((ca,), (cb,)), ((), ())),
                           preferred_element_type=jnp.float32)


def _proj_body(x_ref, w_ref, o_ref):
    o_ref[...] = _mm(x_ref[...], w_ref[...], 1, 1)


def _tc_project(x, w):
    R = x.shape[0]
    return pl.pallas_call(
        _proj_body,
        grid=(R // 1024,),
        in_specs=[pl.BlockSpec((1024, D), lambda i: (i, 0)),
                  pl.BlockSpec((D, D), lambda i: (0, 0))],
        out_specs=pl.BlockSpec((1024, D), lambda i: (i, 0)),
        out_shape=jax.ShapeDtypeStruct((R, D), jnp.float32),
    )(x, w)


def _edst_body(x_ref, wq_ref, o_ref):
    x = x_ref[...]
    for h in range(H):
        o_ref[:, h * D:(h + 1) * D] = _mm(x, wq_ref[h], 1, 0)


def _tc_edst(ent, wq, rows_out):
    return pl.pallas_call(
        _edst_body,
        grid=(10,),
        in_specs=[pl.BlockSpec((1024, D), lambda i: (i, 0)),
                  pl.BlockSpec((H, D, D), lambda i: (0, 0, 0))],
        out_specs=pl.BlockSpec((1024, H * D), lambda i: (i, 0)),
        out_shape=jax.ShapeDtypeStruct((rows_out, H * D), jnp.float32),
    )(ent, wq)


def _esrcp_body(x_ref, wq_ref, fr_ref, o_ref):
    x = x_ref[...]
    for h in range(H):
        o_ref[:, h * D:(h + 1) * D] = _mm(x, wq_ref[h], 1, 0)
    o_ref[:, H * D:] = _mm(x, fr_ref[...], 1, 1)


def _tc_esrcp(ent, wq, frel_pad):
    return pl.pallas_call(
        _esrcp_body,
        grid=(10,),
        in_specs=[pl.BlockSpec((1024, D), lambda i: (i, 0)),
                  pl.BlockSpec((H, D, D), lambda i: (0, 0, 0)),
                  pl.BlockSpec((128, D), lambda i: (0, 0))],
        out_specs=pl.BlockSpec((1024, ESP), lambda i: (i, 0)),
        out_shape=jax.ShapeDtypeStruct((N + NB, ESP), jnp.float32),
    )(ent, wq, frel_pad)


def _merge_body(m_ref, s_ref, mg_ref, inv_ref):
    m = m_ref[...]
    s = s_ref[...]
    mg = jnp.max(m, axis=0, keepdims=True)
    sg = jnp.sum(s * jnp.exp(m - mg), axis=0, keepdims=True)
    mg_ref[...] = mg
    inv_ref[...] = 1.0 / (sg + 1e-16)


def _tc_merge(mtab, stab):
    return pl.pallas_call(
        _merge_body,
        in_specs=[pl.BlockSpec((NW, TW), lambda: (0, 0)),
                  pl.BlockSpec((NW, TW), lambda: (0, 0))],
        out_specs=[pl.BlockSpec((1, TW), lambda: (0, 0)),
                   pl.BlockSpec((1, TW), lambda: (0, 0))],
        out_shape=[jax.ShapeDtypeStruct((1, TW), jnp.float32),
                   jax.ShapeDtypeStruct((1, TW), jnp.float32)],
    )(mtab, stab)


def _iter_body(p_ref, wh_ref, dctx_ref, whd_ref, wq_ref, ef_ref, edst_ref):
    agg = p_ref[0] + p_ref[1]
    dch = _mm(dctx_ref[...], whd_ref[...], 1, 0)
    ef = _mm(agg, wh_ref[...], 1, 0) + dch
    ef_ref[...] = ef
    if edst_ref is not None:
        for h in range(H):
            edst_ref[:, h * D:(h + 1) * D] = _mm(ef, wq_ref[h], 1, 0)


def _tc_iter(partial, wh, dctx, whd, wq, with_q):
    if with_q:
        body = _iter_body
        out_specs = [pl.BlockSpec((1024, D), lambda i: (i, 0)),
                     pl.BlockSpec((1024, H * D), lambda i: (i, 0))]
        out_shape = [jax.ShapeDtypeStruct((N + NB, D), jnp.float32),
                     jax.ShapeDtypeStruct((N + NB, H * D), jnp.float32)]
    else:
        def body(a, b, c, d, e, f):
            return _iter_body(a, b, c, d, e, f, None)
        out_specs = [pl.BlockSpec((1024, D), lambda i: (i, 0))]
        out_shape = [jax.ShapeDtypeStruct((N + NB, D), jnp.float32)]
    return pl.pallas_call(
        body,
        grid=(10,),
        in_specs=[
            pl.BlockSpec((2, 1024, H * D), lambda i: (0, i, 0)),
            pl.BlockSpec((H * D, D), lambda i: (0, 0)),
            pl.BlockSpec((1, D), lambda i: (0, 0)),
            pl.BlockSpec((D, D), lambda i: (0, 0)),
            pl.BlockSpec((H, D, D), lambda i: (0, 0, 0)),
        ],
        out_specs=out_specs,
        out_shape=out_shape,
    )(partial, wh, dctx, whd, wq)


def _csa_body(ef_ref, dctx_ref, owi_ref, seeds_ref, o_ref):
    dcv = _mm(dctx_ref[...], owi_ref[...], 1, 0)
    csa = _mm(dcv, ef_ref[...], 1, 1)
    ids = lax.broadcasted_iota(jnp.int32, (1, N), 1)
    for i in range(32):
        csa = csa + jnp.where(ids == seeds_ref[i], 10000.0, 0.0)
    csa = csa - 10000.0
    m = jnp.max(csa)
    ex = jnp.exp(csa - m)
    o_ref[...] = ex / jnp.sum(ex)


def _tc_csa(ef, dctx, owi, seeds):
    return pl.pallas_call(
        _csa_body,
        in_specs=[
            pl.BlockSpec((N, D), lambda: (0, 0)),
            pl.BlockSpec((1, D), lambda: (0, 0)),
            pl.BlockSpec((D, D), lambda: (0, 0)),
            pl.BlockSpec(memory_space=pltpu.SMEM),
        ],
        out_specs=pl.BlockSpec((1, N), lambda: (0, 0)),
        out_shape=jax.ShapeDtypeStruct((1, N), jnp.float32),
    )(ef, dctx, owi, seeds)


def _comb_body(ap_ref, o_ref):
    o_ref[...] = jnp.sum(ap_ref[...], axis=0, keepdims=True)


def _tc_combine(apart):
    return pl.pallas_call(
        _comb_body,
        in_specs=[pl.BlockSpec((NW, ACCW), lambda: (0, 0))],
        out_specs=pl.BlockSpec((1, ACCW), lambda: (0, 0)),
        out_shape=jax.ShapeDtypeStruct((1, ACCW), jnp.float32),
    )(apart)


def _mesh():
    return plsc.VectorSubcoreMesh(core_axis_name="c", subcore_axis_name="s")


def _lane():
    return lax.broadcasted_iota(jnp.int32, (NB,), 0)


def _bcast(v, j):
    return v[jnp.full((NB,), j, jnp.int32)]


def _allsum(v):
    lane = _lane()
    for sh in (8, 4, 2, 1):
        v = v + v[lane ^ sh]
    return v


def _prefix_ms(keys, lo):
    lane = _lane()
    m = lo
    s = jnp.ones((NB,), jnp.float32)
    for sh in (1, 2, 4, 8):
        idx = jnp.maximum(lane - sh, 0)
        pm = m[idx]
        ps = s[idx]
        same = (keys[idx] == keys) & (lane >= sh)
        mn = jnp.maximum(m, jnp.where(same, pm, BIG))
        s = s * jnp.exp(m - mn) + jnp.where(same, ps * jnp.exp(pm - mn), 0.0)
        m = mn
    return m, s


def _runend(keys):
    lane = _lane()
    nxt = keys[jnp.minimum(lane + 1, NB - 1)]
    return (keys != nxt) | (lane == NB - 1)


def _wid():
    return lax.axis_index("s") * 2 + lax.axis_index("c")


def _dots(rows_v, q_v, j, qoff, extra):
    acc = [jnp.zeros((NB,), jnp.float32) for _ in range(H)]
    for k in range(D // NB):
        sl = pl.ds(k * NB, NB)
        u = rows_v[2 * j, sl] + rows_v[2 * j + 1, sl]
        for h in range(H):
            acc[h] = acc[h] + u * q_v[j, pl.ds(qoff + h * D + k * NB, NB)]
    return [_allsum(a) + (0.0 if extra is None else extra) for a in acc]


def _ainf_body(table_hbm, edst_hbm, idx_hbm, key_hbm, lo_hbm, mt_hbm, st_hbm,
               idx_v, key_v, rows_v, ed_v, lo_v, mt_v, st_v,
               sr0, sr1, se0, se1):
    w = _wid()
    lane = _lane()

    def zinit(i, _):
        sl = pl.ds(i * NB, NB)
        mt_v[sl] = jnp.full((NB,), BIG, jnp.float32)
        st_v[sl] = jnp.zeros((NB,), jnp.float32)
        return 0

    lax.fori_loop(0, TW // NB, zinit, 0)
    srows = (sr0, sr1)

    def fetch_idx(b, q):
        r = w * BPT + b
        pltpu.sync_copy(idx_hbm.at[pl.ds(r, 1)], idx_v.at[pl.ds(q, 1)])
        pltpu.sync_copy(key_hbm.at[pl.ds(r * NB, NB)], key_v.at[q])

    def start_rows(q):
        pltpu.make_async_copy(table_hbm.at[idx_v.at[q]],
                              rows_v.at[q], srows[q]).start()

    def wait_rows(q):
        pltpu.make_async_copy(table_hbm.at[idx_v.at[q]],
                              rows_v.at[q], srows[q]).wait()

    def start_ed(q):
        pltpu.make_async_copy(edst_hbm.at[key_v.at[q]], ed_v, se0).start()

    def wait_ed(q):
        pltpu.make_async_copy(edst_hbm.at[key_v.at[q]], ed_v, se0).wait()

    fetch_idx(0, 0)
    start_rows(0)
    start_ed(0)
    fetch_idx(1, 1)

    def one(b, q):
        r = w * BPT + b

        @pl.when(b + 1 < BPT)
        def _pre():
            start_rows(1 - q)

        wait_rows(q)
        wait_ed(q)
        keys = key_v[q, :]

        def edge(j, lo4):
            ls = _dots(rows_v.at[q], ed_v, j, 0, None)
            jm = lane == j
            return tuple(jnp.where(jm, ls[h], lo4[h]) for h in range(H))

        lo4 = lax.fori_loop(
            0, NB, edge,
            tuple(jnp.zeros((NB,), jnp.float32) for _ in range(H)))

        @pl.when(b + 1 < BPT)
        def _pree():
            start_ed(1 - q)

        valid = (r * NB + lane) < E
        for h in range(H):
            lo = lo4[h]
            lo = jnp.where(lo > 0, lo, NEG * lo)
            lo = jnp.where(valid, lo, BIG)
            lo_v[0, pl.ds(h * NB, NB)] = lo
            pm, ps = _prefix_ms(keys, lo)
            tix = keys * H + h
            mt = plsc.load_gather(mt_v, [tix])
            st = plsc.load_gather(st_v, [tix])
            mn = jnp.maximum(mt, pm)
            sn = st * jnp.exp(mt - mn) + ps * jnp.exp(pm - mn)
            re = _runend(keys)
            plsc.store_scatter(mt_v, [tix], mn, mask=re)
            plsc.store_scatter(st_v, [tix], sn, mask=re)
        pltpu.sync_copy(lo_v, lo_hbm.at[pl.ds(r, 1)])

        @pl.when(b + 2 < BPT)
        def _nidx():
            fetch_idx(b + 2, q)

    def pair(bb, _):
        one(2 * bb, 0)
        one(2 * bb + 1, 1)
        return 0

    lax.fori_loop(0, BPT // 2, pair, 0)
    pltpu.sync_copy(mt_v, mt_hbm.at[w])
    pltpu.sync_copy(st_v, st_hbm.at[w])


def _sc_ainf(table, edst, idx2d, key2d):
    k = functools.partial(
        pl.kernel, mesh=_mesh(),
        compiler_params=pltpu.CompilerParams(needs_layout_passes=False),
        out_type=[jax.ShapeDtypeStruct((EPB, H * NB), jnp.float32),
                  jax.ShapeDtypeStruct((NW, TW), jnp.float32),
                  jax.ShapeDtypeStruct((NW, TW), jnp.float32)],
        scratch_types=[
            pltpu.VMEM((2, 2 * NB), jnp.int32),
            pltpu.VMEM((2, NB), jnp.int32),
            pltpu.VMEM((2, 2 * NB, D), jnp.float32),
            pltpu.VMEM((NB, H * D), jnp.float32),
            pltpu.VMEM((1, H * NB), jnp.float32),
            pltpu.VMEM((TW,), jnp.float32),
            pltpu.VMEM((TW,), jnp.float32),
            pltpu.SemaphoreType.DMA,
            pltpu.SemaphoreType.DMA,
            pltpu.SemaphoreType.DMA,
            pltpu.SemaphoreType.DMA,
        ],
    )(_ainf_body)
    return k(table, edst, idx2d, key2d)


def _binf_body(table_hbm, idx_hbm, key_hbm, lo_hbm, mg_hbm, inv_hbm, out_hbm,
               idx_v, key_v, rows_v, lo_v, mt_v, iv_v, acc_v, stage_v,
               tgt_v, sr0, sr1):
    w = _wid()
    sc = lax.axis_index("c")
    sid = lax.axis_index("s")
    lane = _lane()

    def zstage(i, _):
        for k in range(H * D // NB):
            stage_v[i, pl.ds(k * NB, NB)] = jnp.zeros((NB,), jnp.float32)
        return 0

    lax.fori_loop(0, NB, zstage, 0)

    def zrow(i, _):
        pltpu.sync_copy(
            stage_v,
            out_hbm.at[pl.ds(sc * PCH + sid * 640 + i * NB, NB)])
        return 0

    lax.fori_loop(0, 40, zrow, 0)
    pltpu.sync_copy(mg_hbm, mt_v)
    pltpu.sync_copy(inv_hbm, iv_v)

    def z2(i, _):
        acc_v[0, pl.ds(i * NB, NB)] = jnp.zeros((NB,), jnp.float32)
        return 0

    lax.fori_loop(0, H * D // NB, z2, 0)
    plsc.subcore_barrier()

    srows = (sr0, sr1)

    def fetch_idx(b, q):
        r = w * BPT + b
        pltpu.sync_copy(idx_hbm.at[pl.ds(r, 1)], idx_v.at[pl.ds(q, 1)])
        pltpu.sync_copy(key_hbm.at[pl.ds(r * NB, NB)], key_v.at[q])
        pltpu.sync_copy(lo_hbm.at[pl.ds(r, 1)], lo_v.at[pl.ds(q, 1)])

    def start_rows(q):
        pltpu.make_async_copy(table_hbm.at[idx_v.at[q]],
                              rows_v.at[q], srows[q]).start()

    def wait_rows(q):
        pltpu.make_async_copy(table_hbm.at[idx_v.at[q]],
                              rows_v.at[q], srows[q]).wait()

    fetch_idx(0, 0)
    start_rows(0)
    fetch_idx(1, 1)

    def batch(b, q, prevkey):
        r = w * BPT + b

        @pl.when(b + 1 < BPT)
        def _pre():
            start_rows(1 - q)

        wait_rows(q)
        rows_q = rows_v.at[q]
        keys = key_v[q, :]
        att = []
        for h in range(H):
            lo = lo_v[q, pl.ds(h * NB, NB)]
            tix = keys * H + h
            m = plsc.load_gather(mt_v, [tix])
            iv = plsc.load_gather(iv_v, [tix])
            att.append(jnp.exp(lo - m) * iv)
        prev0 = _bcast(prevkey, NB - 1)

        def edge(j, _):
            kj = keys[jnp.full((NB,), j, jnp.int32)]
            km = keys[jnp.maximum(jnp.full((NB,), j, jnp.int32) - 1, 0)]
            isz = jnp.full((NB,), j, jnp.int32) == 0
            pj = kj == jnp.where(isz, prev0, km)
            zf = pj.astype(jnp.float32)
            aj = [att[h][jnp.full((NB,), j, jnp.int32)] for h in range(H)]
            for k in range(D // NB):
                sl = pl.ds(k * NB, NB)
                u = rows_q[2 * j, sl] + rows_q[2 * j + 1, sl]
                for h in range(H):
                    osl = pl.ds(h * D + k * NB, NB)
                    nv = acc_v[0, osl] * zf + u * aj[h]
                    acc_v[0, osl] = nv
                    stage_v[j, osl] = nv
            return 0

        lax.fori_loop(0, NB, edge, 0)
        re = _runend(keys)
        tgt_v[0, pl.ds(0, NB)] = (sc * PCH
                                  + jnp.where(re, keys, NPP + 128 + lane))
        pltpu.sync_copy(stage_v, out_hbm.at[tgt_v.at[0]])

        @pl.when(b + 2 < BPT)
        def _nidx():
            fetch_idx(b + 2, q)

        return keys

    def pair(bb, prevkey):
        k0 = batch(2 * bb, 0, prevkey)
        return batch(2 * bb + 1, 1, k0)

    lax.fori_loop(0, BPT // 2, pair, jnp.full((NB,), -1, jnp.int32))


def _sc_binf(table, idx2d, key2d, lo, mg, inv):
    k = functools.partial(
        pl.kernel, mesh=_mesh(),
        compiler_params=pltpu.CompilerParams(needs_layout_passes=False),
        out_type=jax.ShapeDtypeStruct((2 * PCH, H * D), jnp.float32),
        scratch_types=[
            pltpu.VMEM((2, 2 * NB), jnp.int32),
            pltpu.VMEM((2, NB), jnp.int32),
            pltpu.VMEM((2, 2 * NB, D), jnp.float32),
            pltpu.VMEM((2, H * NB), jnp.float32),
            pltpu.VMEM((TW,), jnp.float32),
            pltpu.VMEM((TW,), jnp.float32),
            pltpu.VMEM((1, H * D), jnp.float32),
            pltpu.VMEM((NB, H * D), jnp.float32),
            pltpu.VMEM((1, NB), jnp.int32),
            pltpu.SemaphoreType.DMA,
            pltpu.SemaphoreType.DMA,
        ],
    )(_binf_body)
    return k(table, idx2d, key2d, lo, mg, inv)


def _aout_body(table_hbm, esp_hbm, idx_hbm, key_hbm, typ_hbm,
               lo_hbm, mt_hbm, st_hbm,
               idx_v, key_v, typ_v, rows_v, es_v, lo_v, mt_v, st_v,
               sr0, sr1, se0, se1):
    w = _wid()
    lane = _lane()

    def zinit(i, _):
        sl = pl.ds(i * NB, NB)
        mt_v[sl] = jnp.full((NB,), BIG, jnp.float32)
        st_v[sl] = jnp.zeros((NB,), jnp.float32)
        return 0

    lax.fori_loop(0, TW // NB, zinit, 0)
    srows = (sr0, sr1)

    def fetch_idx(b, q):
        r = w * BPT + b
        pltpu.sync_copy(idx_hbm.at[pl.ds(r, 1)], idx_v.at[pl.ds(q, 1)])
        pltpu.sync_copy(key_hbm.at[pl.ds(r * NB, NB)], key_v.at[q])
        pltpu.sync_copy(typ_hbm.at[pl.ds(r * NB, NB)], typ_v.at[q])

    def start_rows(q):
        pltpu.make_async_copy(table_hbm.at[idx_v.at[q]],
                              rows_v.at[q], srows[q]).start()

    def wait_rows(q):
        pltpu.make_async_copy(table_hbm.at[idx_v.at[q]],
                              rows_v.at[q], srows[q]).wait()

    def start_es(q):
        pltpu.make_async_copy(esp_hbm.at[key_v.at[q]], es_v, se0).start()

    def wait_es(q):
        pltpu.make_async_copy(esp_hbm.at[key_v.at[q]], es_v, se0).wait()

    fetch_idx(0, 0)
    start_rows(0)
    start_es(0)
    fetch_idx(1, 1)

    def one(b, q):
        r = w * BPT + b

        @pl.when(b + 1 < BPT)
        def _pre():
            start_rows(1 - q)

        wait_rows(q)
        wait_es(q)
        keys = key_v[q, :]
        types = typ_v[q, :]
        esq = es_v
        rowsq = rows_v.at[q]

        def edge(j, lo4):
            rej = plsc.load_gather(
                esq, [jnp.full((NB,), j, jnp.int32),
                      jnp.full((NB,), H * D, jnp.int32) + _bcast(types, j)])
            acc = [jnp.zeros((NB,), jnp.float32) for _ in range(H)]
            for k in range(D // NB):
                sl = pl.ds(k * NB, NB)
                ed = rowsq[j, sl]
                for h in range(H):
                    acc[h] = acc[h] + ed * esq[j, pl.ds(h * D + k * NB, NB)]
            jm = lane == j
            return tuple(jnp.where(jm, _allsum(acc[h]) + rej, lo4[h])
                         for h in range(H))

        lo4 = lax.fori_loop(
            0, NB, edge,
            tuple(jnp.zeros((NB,), jnp.float32) for _ in range(H)))

        @pl.when(b + 1 < BPT)
        def _prees():
            start_es(1 - q)

        valid = (r * NB + lane) < E
        for h in range(H):
            lo = lo4[h]
            lo = jnp.where(lo > 0, lo, NEG * lo)
            lo = jnp.where(valid, lo, BIG)
            lo_v[0, pl.ds(h * NB, NB)] = lo
            pm, ps = _prefix_ms(keys, lo)
            tix = keys * H + h
            mt = plsc.load_gather(mt_v, [tix])
            st = plsc.load_gather(st_v, [tix])
            mn = jnp.maximum(mt, pm)
            sn = st * jnp.exp(mt - mn) + ps * jnp.exp(pm - mn)
            re = _runend(keys)
            plsc.store_scatter(mt_v, [tix], mn, mask=re)
            plsc.store_scatter(st_v, [tix], sn, mask=re)
        pltpu.sync_copy(lo_v, lo_hbm.at[pl.ds(r, 1)])

        @pl.when(b + 2 < BPT)
        def _nidx():
            fetch_idx(b + 2, q)

    def pair(bb, _):
        one(2 * bb, 0)
        one(2 * bb + 1, 1)
        return 0

    lax.fori_loop(0, BPT // 2, pair, 0)
    pltpu.sync_copy(mt_v, mt_hbm.at[w])
    pltpu.sync_copy(st_v, st_hbm.at[w])


def _sc_aout(table, esp, idxo2d, key2d, typ2d):
    k = functools.partial(
        pl.kernel, mesh=_mesh(),
        compiler_params=pltpu.CompilerParams(needs_layout_passes=False),
        out_type=[jax.ShapeDtypeStruct((EPB, H * NB), jnp.float32),
                  jax.ShapeDtypeStruct((NW, TW), jnp.float32),
                  jax.ShapeDtypeStruct((NW, TW), jnp.float32)],
        scratch_types=[
            pltpu.VMEM((2, NB), jnp.int32),
            pltpu.VMEM((2, NB), jnp.int32),
            pltpu.VMEM((2, NB), jnp.int32),
            pltpu.VMEM((2, NB, D), jnp.float32),
            pltpu.VMEM((NB, ESP), jnp.float32),
            pltpu.VMEM((1, H * NB), jnp.float32),
            pltpu.VMEM((TW,), jnp.float32),
            pltpu.VMEM((TW,), jnp.float32),
            pltpu.SemaphoreType.DMA,
            pltpu.SemaphoreType.DMA,
            pltpu.SemaphoreType.DMA,
            pltpu.SemaphoreType.DMA,
        ],
    )(_aout_body)
    return k(table, esp, idxo2d, key2d, typ2d)


def _bout_body(lo_hbm, key_hbm, dst_hbm, a_hbm, mg_hbm, inv_hbm, ap_hbm,
               key_v, dst_v, lo_v, mt_v, iv_v, a_v, acc_v, sem):
    w = _wid()
    lane = _lane()
    pltpu.sync_copy(mg_hbm, mt_v)
    pltpu.sync_copy(inv_hbm, iv_v)
    pltpu.sync_copy(a_hbm, a_v)

    def z(i, _):
        acc_v[pl.ds(i * NB, NB)] = jnp.zeros((NB,), jnp.float32)
        return 0

    lax.fori_loop(0, ACCW // NB, z, 0)

    def batch(b, _):
        r = w * BPT + b
        pltpu.sync_copy(key_hbm.at[pl.ds(r * NB, NB)], key_v)
        pltpu.sync_copy(dst_hbm.at[pl.ds(r * NB, NB)], dst_v)
        pltpu.sync_copy(lo_hbm.at[pl.ds(r, 1)], lo_v)
        keys = key_v[...]
        dsts = dst_v[...]
        tp = jnp.zeros((NB,), jnp.float32)
        for h in range(H):
            lo = lo_v[0, pl.ds(h * NB, NB)]
            tix = keys * H + h
            m = plsc.load_gather(mt_v, [tix])
            iv = plsc.load_gather(iv_v, [tix])
            tp = tp + jnp.exp(lo - m) * iv
        asrc = plsc.load_gather(a_v, [keys])
        valid = (r * NB + lane) < E
        c = jnp.where(valid, tp * asrc * (1.0 / H), 0.0)
        plsc.addupdate_scatter(acc_v, [dsts], c, mask=valid)
        return 0

    lax.fori_loop(0, BPT, batch, 0)
    pltpu.sync_copy(acc_v, ap_hbm.at[w])


def _sc_bout(lo, key2d, dst2d, a_in, mg, inv):
    k = functools.partial(
        pl.kernel, mesh=_mesh(),
        compiler_params=pltpu.CompilerParams(needs_layout_passes=False),
        out_type=jax.ShapeDtypeStruct((NW, ACCW), jnp.float32),
        scratch_types=[
            pltpu.VMEM((NB,), jnp.int32),
            pltpu.VMEM((NB,), jnp.int32),
            pltpu.VMEM((1, H * NB), jnp.float32),
            pltpu.VMEM((TW,), jnp.float32),
            pltpu.VMEM((TW,), jnp.float32),
            pltpu.VMEM((ACCW,), jnp.float32),
            pltpu.VMEM((ACCW,), jnp.float32),
            pltpu.SemaphoreType.DMA,
        ],
    )(_bout_body)
    return k(lo, key2d, dst2d, a_in, mg, inv)


def kernel(entity_emb, relation_emb, fc_w, w_q, w_h_entity, w_h_dialogue,
           out_w_init, out_w_q, dialogue_context, node_ids, edge_types,
           edge_index, seed_set):
    i32 = jnp.int32
    src = edge_index[0]
    dst = edge_index[1]
    seeds = seed_set.astype(i32)

    perm_d = jnp.argsort(dst)
    srcd = src[perm_d].astype(i32)
    td = edge_types[perm_d].astype(i32)
    dstd = dst[perm_d].astype(i32)
    perm_s = jnp.argsort(src)
    dsts = dst[perm_s].astype(i32)
    ts = edge_types[perm_s].astype(i32)
    srcs = src[perm_s].astype(i32)

    pad = EP - E
    idx_in = jnp.stack([srcd, N + td], 1).reshape(-1)
    idx_in = jnp.pad(idx_in, (0, 2 * pad)).reshape(EPB, 2 * NB)
    dstd_p = jnp.pad(dstd, (0, pad), constant_values=N)
    srcs_p = jnp.pad(srcs, (0, pad), constant_values=N)
    dsts_p = jnp.pad(dsts, (0, pad), constant_values=N)
    ts_p = jnp.pad(ts, (0, pad))
    idx_out = jnp.pad(dsts, (0, pad)).reshape(EPB, NB)

    X = jnp.concatenate([entity_emb, relation_emb], 0)
    Xp = jnp.pad(X, ((0, TPAD - NT), (0, 0)))
    table0 = _tc_project(Xp, fc_w)
    suffix = table0[N:]
    frel_pad = jnp.pad(table0[N:NT], ((0, 128 - NREL), (0, 0)))
    dctx = dialogue_context

    def inflow(table_i, edst_i):
        lo, mtab, stab = _sc_ainf(table_i, edst_i, idx_in, dstd_p)
        mg, inv = _tc_merge(mtab, stab)
        part = _sc_binf(table_i, idx_in, dstd_p, lo, mg.reshape(-1),
                        inv.reshape(-1))
        return part.reshape(2, PCH, H * D)

    def outflow(table_i, esp_i, a_in):
        lo, mtab, stab = _sc_aout(table_i, esp_i, idx_out, srcs_p, ts_p)
        mg, inv = _tc_merge(mtab, stab)
        ap = _sc_bout(lo, srcs_p, dsts_p, a_in, mg.reshape(-1),
                      inv.reshape(-1))
        return _tc_combine(ap).reshape(-1)

    edst0 = _tc_edst(table0[:N + NB], w_q, N + NB)
    part = inflow(table0, edst0)
    ef1, edst1 = _tc_iter(part, w_h_entity, dctx, w_h_dialogue, w_q, True)
    table1 = jnp.concatenate([ef1[:N], suffix], 0)
    part = inflow(table1, edst1)
    ef2, edst2 = _tc_iter(part, w_h_entity, dctx, w_h_dialogue, w_q, True)
    table2 = jnp.concatenate([ef2[:N], suffix], 0)
    part = inflow(table2, edst2)
    (ef3,) = _tc_iter(part, w_h_entity, dctx, w_h_dialogue, w_q, False)
    table3 = jnp.concatenate([ef3[:N], suffix], 0)

    a0 = _tc_csa(ef1[:N], dctx, out_w_init, seeds)
    a0p = jnp.pad(a0.reshape(-1), (0, ACCW - N))

    esp1 = _tc_esrcp(table2[:N + NB], out_w_q, frel_pad)
    a1 = outflow(table2, esp1, a0p)
    esp2 = _tc_esrcp(table3[:N + NB], out_w_q, frel_pad)
    a2 = outflow(table3, esp2, a1)
    return a2[:N]
